# baseline (speedup 1.0000x reference)
import functools

import jax
import jax.numpy as jnp
from jax import lax
from jax.experimental import pallas as pl
from jax.experimental.pallas import tpu as pltpu
from jax.experimental.pallas import tpu_sc as plsc

NN = 10000
NP = 10240
NE = 160000
EP = 163840
D = 256
DH = 128
NSUB = 16
NCORE = 2
K = 128
EPS = EP // NSUB
EPW = EP // (NSUB * NCORE)
STRIPE = NP // NSUB

_mesh = plsc.VectorSubcoreMesh(core_axis_name="c", subcore_axis_name="s")


NCHD = EPW // K
NCH = EPS // K


@functools.partial(
    pl.kernel,
    mesh=_mesh,
    out_type=jax.ShapeDtypeStruct((NCORE * NP,), jnp.float32),
    scratch_types=[
        pltpu.VMEM((NCHD, K), jnp.int32),
        pltpu.VMEM((K,), jnp.float32),
        pltpu.VMEM_SHARED((NP,), jnp.float32),
        pltpu.SemaphoreType.DMA,
    ],
)
def _degree_kernel(dst_hbm, zeros_hbm, out_hbm, idx_v, ones_v, acc_sh, ssem):
    cid = lax.axis_index("c")
    sid = lax.axis_index("s")
    wid = cid * NSUB + sid
    pltpu.sync_copy(dst_hbm.at[wid], idx_v)
    for j in range(K // 16):
        ones_v[pl.ds(16 * j, 16)] = jnp.ones((16,), jnp.float32)

    @pl.when(sid == 0)
    def _zero():
        pltpu.sync_copy(zeros_hbm, acc_sh)

    plsc.subcore_barrier()

    def fire(i, carry):
        pltpu.async_copy(ones_v, acc_sh.at[idx_v.at[i]], ssem, add=True)
        return carry

    def drain(i, carry):
        pltpu.make_async_copy(ones_v, acc_sh.at[idx_v.at[0]], ssem).wait()
        return carry

    lax.fori_loop(0, NCHD, fire, 0)
    lax.fori_loop(0, NCHD, drain, 0)
    plsc.subcore_barrier()
    pltpu.sync_copy(
        acc_sh.at[pl.ds(sid * STRIPE, STRIPE)],
        out_hbm.at[pl.ds(cid * NP + sid * STRIPE, STRIPE)],
    )


@functools.partial(
    pl.kernel,
    mesh=_mesh,
    out_type=jax.ShapeDtypeStruct((NCORE * NP, DH), jnp.float32),
    scratch_types=[
        pltpu.VMEM((64,), jnp.int32),
        pltpu.VMEM((64,), jnp.int32),
        pltpu.VMEM((NCH, K), jnp.int32),
        pltpu.VMEM((64, 256), jnp.float32),
        pltpu.VMEM((64, 256), jnp.float32),
        pltpu.VMEM_SHARED((NP, DH), jnp.float32),
        pltpu.SemaphoreType.DMA,
        pltpu.SemaphoreType.DMA,
        pltpu.SemaphoreType.DMA,
        pltpu.SemaphoreType.DMA,
        pltpu.SemaphoreType.DMA,
        pltpu.SemaphoreType.DMA,
    ],
)
def _segsum_kernel(y_hbm, srcs_hbm, dst_hbm, zrows_hbm, out_hbm,
                   src0, src1, dst_v, rows0, rows1,
                   acc_sh, isem0, isem1, gsem0, gsem1, ssem0, ssem1):
    cid = lax.axis_index("c")
    sid = lax.axis_index("s")
    wid = cid * NSUB + sid
    sbase = wid * EPS
    pltpu.sync_copy(dst_hbm.at[sid], dst_v)
    pltpu.sync_copy(zrows_hbm, acc_sh.at[pl.ds(sid * STRIPE, STRIPE), :])
    plsc.subcore_barrier()
    bufs = ((src0, isem0, rows0, gsem0, ssem0),
            (src1, isem1, rows1, gsem1, ssem1))

    def start_idx(j, b):
        sv, isem = bufs[b][0], bufs[b][1]
        j = jnp.minimum(j, NCH - 1)
        pltpu.async_copy(srcs_hbm.at[pl.ds(sbase + j * K, 64)], sv, isem)

    def wait_idx(b):
        sv, isem = bufs[b][0], bufs[b][1]
        pltpu.make_async_copy(srcs_hbm.at[pl.ds(sbase, 64)], sv, isem).wait()

    def start_gather(b):
        sv, rv, gsem = bufs[b][0], bufs[b][2], bufs[b][3]
        pltpu.async_copy(y_hbm.at[sv], rv, gsem)

    def wait_gather(b):
        sv, rv, gsem = bufs[b][0], bufs[b][2], bufs[b][3]
        pltpu.make_async_copy(y_hbm.at[sv], rv, gsem).wait()

    def start_scatter(i, b):
        pass

    def wait_scatter(b):
        pass

    start_idx(0, 0)
    start_idx(1, 1)
    wait_idx(0)
    start_gather(0)
    wait_gather(0)
    start_scatter(0, 0)
    start_idx(2, 0)
    wait_idx(1)
    start_gather(1)

    def step(i, b):
        nb = 1 - b
        wait_gather(b)
        start_scatter(i, b)
        start_idx(i + 2, b)
        wait_scatter(nb)
        wait_idx(nb)
        start_gather(nb)

    def pair(g, carry):
        step(2 * g + 1, 1)
        step(2 * g + 2, 0)
        return carry

    lax.fori_loop(0, (NCH - 2) // 2, pair, 0)
    wait_gather(1)
    start_scatter(NCH - 1, 1)
    wait_scatter(0)
    wait_scatter(1)
    wait_idx(0)
    plsc.subcore_barrier()
    pltpu.sync_copy(
        acc_sh.at[pl.ds(sid * STRIPE, STRIPE), :],
        out_hbm.at[pl.ds(cid * NP + sid * STRIPE, STRIPE), :],
    )


_RB = 1280


def _mm_scale_body(x_ref, w_ref, p0_ref, p1_ref, y_ref):
    dinv = lax.rsqrt(p0_ref[...] + p1_ref[...] + 1.0)
    y = jnp.dot(x_ref[...], w_ref[...], preferred_element_type=jnp.float32)
    y = y * dinv
    y_ref[0] = y[:, :DH]
    y_ref[1] = y[:, DH:]


def _mm_scale(xp, w, p0, p1):
    return pl.pallas_call(
        _mm_scale_body,
        grid=(NP // _RB,),
        in_specs=[
            pl.BlockSpec((_RB, D), lambda i: (i, 0)),
            pl.BlockSpec((D, D), lambda i: (0, 0)),
            pl.BlockSpec((_RB, 1), lambda i: (i, 0)),
            pl.BlockSpec((_RB, 1), lambda i: (i, 0)),
        ],
        out_specs=pl.BlockSpec((2, _RB, DH), lambda i: (0, i, 0)),
        out_shape=jax.ShapeDtypeStruct((2, NP, DH), jnp.float32),
    )(xp, w, p0, p1)


def _combine_mm_body(s_ref, y_ref, p0_ref, p1_ref, b_ref, w_ref, o_ref):
    dinv = lax.rsqrt(p0_ref[...] + p1_ref[...] + 1.0)
    b = b_ref[...]
    h0 = jnp.maximum((s_ref[0] + y_ref[0]) * dinv + b[:, :DH], 0.0)
    h1 = jnp.maximum((s_ref[1] + y_ref[1]) * dinv + b[:, DH:], 0.0)
    y2 = jnp.dot(h0, w_ref[:DH, :], preferred_element_type=jnp.float32)
    y2 = y2 + jnp.dot(h1, w_ref[DH:, :], preferred_element_type=jnp.float32)
    y2 = y2 * dinv
    o_ref[0] = y2[:, :DH]
    o_ref[1] = y2[:, DH:]


def _combine_mm(s, y, p0, p1, b, w):
    return pl.pallas_call(
        _combine_mm_body,
        grid=(NP // _RB,),
        in_specs=[
            pl.BlockSpec((2, _RB, DH), lambda i: (0, i, 0)),
            pl.BlockSpec((2, _RB, DH), lambda i: (0, i, 0)),
            pl.BlockSpec((_RB, 1), lambda i: (i, 0)),
            pl.BlockSpec((_RB, 1), lambda i: (i, 0)),
            pl.BlockSpec((1, D), lambda i: (0, 0)),
            pl.BlockSpec((D, D), lambda i: (0, 0)),
        ],
        out_specs=pl.BlockSpec((2, _RB, DH), lambda i: (0, i, 0)),
        out_shape=jax.ShapeDtypeStruct((2, NP, DH), jnp.float32),
    )(s, y, p0, p1, b, w)


def _final_body(s_ref, y_ref, p0_ref, p1_ref, b_ref, o_ref):
    dinv = lax.rsqrt(p0_ref[...] + p1_ref[...] + 1.0)
    b = b_ref[...]
    o_ref[:, :DH] = (s_ref[0] + y_ref[0]) * dinv + b[:, :DH]
    o_ref[:, DH:] = (s_ref[1] + y_ref[1]) * dinv + b[:, DH:]


def _final(s, y, p0, p1, b):
    return pl.pallas_call(
        _final_body,
        grid=(NP // _RB,),
        in_specs=[
            pl.BlockSpec((2, _RB, DH), lambda i: (0, i, 0)),
            pl.BlockSpec((2, _RB, DH), lambda i: (0, i, 0)),
            pl.BlockSpec((_RB, 1), lambda i: (i, 0)),
            pl.BlockSpec((_RB, 1), lambda i: (i, 0)),
            pl.BlockSpec((1, D), lambda i: (0, 0)),
        ],
        out_specs=pl.BlockSpec((_RB, D), lambda i: (i, 0)),
        out_shape=jax.ShapeDtypeStruct((NP, D), jnp.float32),
    )(s, y, p0, p1, b)


def kernel(X, edges, W1, b1, W2, b2):
    src = edges[0].astype(jnp.int32)
    dst = edges[1].astype(jnp.int32)
    npad = EP - NE
    pad_idx = NN + (jnp.arange(npad, dtype=jnp.int32) % (NP - NN))
    src_p = jnp.concatenate([src, pad_idx])
    dst_p = jnp.concatenate([dst, pad_idx])
    srcs2 = jnp.concatenate([src_p, src_p])
    dst3 = dst_p.reshape(NSUB, NCH, K)
    dst_deg = dst_p.reshape(2 * NSUB, NCHD, K)

    xp = jnp.pad(X, ((0, NP - NN), (0, 0)))
    zeros_np = jnp.zeros((NP,), jnp.float32)
    zrows = jnp.zeros((STRIPE, DH), jnp.float32)
    b1r = b1.reshape(1, D)
    b2r = b2.reshape(1, D)

    degf = _degree_kernel(dst_deg, zeros_np)
    p0 = degf[:NP].reshape(NP, 1)
    p1 = degf[NP:].reshape(NP, 1)

    y1 = _mm_scale(xp, W1, p0, p1)
    s1 = _segsum_kernel(y1.reshape(NP, 2 * DH), srcs2, dst3, zrows)
    y2 = _combine_mm(s1.reshape(2, NP, DH), y1, p0, p1, b1r, W2)
    s2 = _segsum_kernel(y2.reshape(NP, 2 * DH), srcs2, dst3, zrows)
    out = _final(s2.reshape(2, NP, DH), y2, p0, p1, b2r)
    return out[:NN]


# speedup vs baseline: 1.1145x; 1.1145x over previous
import functools

import jax
import jax.numpy as jnp
from jax import lax
from jax.experimental import pallas as pl
from jax.experimental.pallas import tpu as pltpu
from jax.experimental.pallas import tpu_sc as plsc

NN = 10000
NP = 10240
NE = 160000
EP = 163840
D = 256
DH = 128
NSUB = 16
NCORE = 2
K = 128
EPS = EP // NSUB
EPW = EP // (NSUB * NCORE)
STRIPE = NP // NSUB

_mesh = plsc.VectorSubcoreMesh(core_axis_name="c", subcore_axis_name="s")


NCHD = EPW // K
NCH = EPS // K


@functools.partial(
    pl.kernel,
    mesh=_mesh,
    out_type=jax.ShapeDtypeStruct((NCORE * NP,), jnp.float32),
    scratch_types=[
        pltpu.VMEM((NCHD, K), jnp.int32),
        pltpu.VMEM((K,), jnp.float32),
        pltpu.VMEM_SHARED((NP,), jnp.float32),
        pltpu.SemaphoreType.DMA,
    ],
)
def _degree_kernel(dst_hbm, zeros_hbm, out_hbm, idx_v, ones_v, acc_sh, ssem):
    cid = lax.axis_index("c")
    sid = lax.axis_index("s")
    wid = cid * NSUB + sid
    pltpu.sync_copy(dst_hbm.at[wid], idx_v)
    for j in range(K // 16):
        ones_v[pl.ds(16 * j, 16)] = jnp.ones((16,), jnp.float32)

    @pl.when(sid == 0)
    def _zero():
        pltpu.sync_copy(zeros_hbm, acc_sh)

    plsc.subcore_barrier()

    def fire(i, carry):
        pltpu.async_copy(ones_v, acc_sh.at[idx_v.at[i]], ssem, add=True)
        return carry

    def drain(i, carry):
        pltpu.make_async_copy(ones_v, acc_sh.at[idx_v.at[0]], ssem).wait()
        return carry

    lax.fori_loop(0, NCHD, fire, 0)
    lax.fori_loop(0, NCHD, drain, 0)
    plsc.subcore_barrier()
    pltpu.sync_copy(
        acc_sh.at[pl.ds(sid * STRIPE, STRIPE)],
        out_hbm.at[pl.ds(cid * NP + sid * STRIPE, STRIPE)],
    )


@functools.partial(
    pl.kernel,
    mesh=_mesh,
    out_type=jax.ShapeDtypeStruct((NCORE * NP, DH), jnp.float32),
    scratch_types=[
        pltpu.VMEM((K,), jnp.int32),
        pltpu.VMEM((K,), jnp.int32),
        pltpu.VMEM((NCH, K), jnp.int32),
        pltpu.VMEM((K, DH), jnp.float32),
        pltpu.VMEM((K, DH), jnp.float32),
        pltpu.VMEM_SHARED((NP, DH), jnp.float32),
        pltpu.SemaphoreType.DMA,
        pltpu.SemaphoreType.DMA,
        pltpu.SemaphoreType.DMA,
        pltpu.SemaphoreType.DMA,
        pltpu.SemaphoreType.DMA,
        pltpu.SemaphoreType.DMA,
    ],
)
def _segsum_kernel(y_hbm, srcs_hbm, dst_hbm, out_hbm,
                   src0, src1, dst_v, rows0, rows1,
                   acc_sh, isem0, isem1, gsem0, gsem1, ssem0, ssem1):
    cid = lax.axis_index("c")
    sid = lax.axis_index("s")
    wid = cid * NSUB + sid
    sbase = wid * EPS
    pltpu.sync_copy(dst_hbm.at[sid], dst_v)
    zv = jnp.zeros((16,), jnp.float32)

    def _zrow(r, carry):
        for c in range(DH // 16):
            rows0[r, pl.ds(c * 16, 16)] = zv
        return carry

    lax.fori_loop(0, K, _zrow, 0)
    for t in range(STRIPE // K):
        pltpu.sync_copy(rows0, acc_sh.at[pl.ds(sid * STRIPE + t * K, K), :])
    plsc.subcore_barrier()
    bufs = ((src0, isem0, rows0, gsem0, ssem0),
            (src1, isem1, rows1, gsem1, ssem1))

    def start_idx(j, b):
        sv, isem = bufs[b][0], bufs[b][1]
        j = jnp.minimum(j, NCH - 1)
        pltpu.async_copy(srcs_hbm.at[pl.ds(sbase + j * K, K)], sv, isem)

    def wait_idx(b):
        sv, isem = bufs[b][0], bufs[b][1]
        pltpu.make_async_copy(srcs_hbm.at[pl.ds(sbase, K)], sv, isem).wait()

    def start_gather(b):
        sv, rv, gsem = bufs[b][0], bufs[b][2], bufs[b][3]
        pltpu.async_copy(y_hbm.at[sv], rv, gsem)

    def wait_gather(b):
        sv, rv, gsem = bufs[b][0], bufs[b][2], bufs[b][3]
        pltpu.make_async_copy(y_hbm.at[sv], rv, gsem).wait()

    def start_scatter(i, b):
        rv, ssem = bufs[b][2], bufs[b][4]
        pltpu.async_copy(rv, acc_sh.at[dst_v.at[i]], ssem, add=True)

    def wait_scatter(b):
        rv, ssem = bufs[b][2], bufs[b][4]
        pltpu.make_async_copy(rv, acc_sh.at[dst_v.at[0]], ssem).wait()

    start_idx(0, 0)
    start_idx(1, 1)
    wait_idx(0)
    start_gather(0)
    wait_gather(0)
    start_scatter(0, 0)
    start_idx(2, 0)
    wait_idx(1)
    start_gather(1)

    def step(i, b):
        nb = 1 - b
        wait_gather(b)
        start_scatter(i, b)
        start_idx(i + 2, b)
        wait_scatter(nb)
        wait_idx(nb)
        start_gather(nb)

    def pair(g, carry):
        step(2 * g + 1, 1)
        step(2 * g + 2, 0)
        return carry

    lax.fori_loop(0, (NCH - 2) // 2, pair, 0)
    wait_gather(1)
    start_scatter(NCH - 1, 1)
    wait_scatter(0)
    wait_scatter(1)
    wait_idx(0)
    plsc.subcore_barrier()
    pltpu.sync_copy(
        acc_sh.at[pl.ds(sid * STRIPE, STRIPE), :],
        out_hbm.at[pl.ds(cid * NP + sid * STRIPE, STRIPE), :],
    )


_RB = 1000
_NG = NN // _RB


def _mm_scale_body(x_ref, w_ref, p0_ref, p1_ref, y_ref):
    dinv = lax.rsqrt(p0_ref[...] + p1_ref[...] + 1.0)
    y = jnp.dot(x_ref[...], w_ref[...], preferred_element_type=jnp.float32)
    y = y * dinv
    y_ref[0] = y[:, :DH]
    y_ref[1] = y[:, DH:]


def _mm_scale(xp, w, p0, p1):
    return pl.pallas_call(
        _mm_scale_body,
        grid=(_NG,),
        in_specs=[
            pl.BlockSpec((_RB, D), lambda i: (i, 0)),
            pl.BlockSpec((D, D), lambda i: (0, 0)),
            pl.BlockSpec((_RB, 1), lambda i: (i, 0)),
            pl.BlockSpec((_RB, 1), lambda i: (i, 0)),
        ],
        out_specs=pl.BlockSpec((2, _RB, DH), lambda i: (0, i, 0)),
        out_shape=jax.ShapeDtypeStruct((2, NP, DH), jnp.float32),
    )(xp, w, p0, p1)


def _combine_mm_body(s_ref, y_ref, p0_ref, p1_ref, b_ref, w_ref, o_ref):
    dinv = lax.rsqrt(p0_ref[...] + p1_ref[...] + 1.0)
    b = b_ref[...]
    h0 = jnp.maximum((s_ref[0] + y_ref[0]) * dinv + b[:, :DH], 0.0)
    h1 = jnp.maximum((s_ref[1] + y_ref[1]) * dinv + b[:, DH:], 0.0)
    y2 = jnp.dot(h0, w_ref[:DH, :], preferred_element_type=jnp.float32)
    y2 = y2 + jnp.dot(h1, w_ref[DH:, :], preferred_element_type=jnp.float32)
    y2 = y2 * dinv
    o_ref[0] = y2[:, :DH]
    o_ref[1] = y2[:, DH:]


def _combine_mm(s, y, p0, p1, b, w):
    return pl.pallas_call(
        _combine_mm_body,
        grid=(_NG,),
        in_specs=[
            pl.BlockSpec((2, _RB, DH), lambda i: (0, i, 0)),
            pl.BlockSpec((2, _RB, DH), lambda i: (0, i, 0)),
            pl.BlockSpec((_RB, 1), lambda i: (i, 0)),
            pl.BlockSpec((_RB, 1), lambda i: (i, 0)),
            pl.BlockSpec((1, D), lambda i: (0, 0)),
            pl.BlockSpec((D, D), lambda i: (0, 0)),
        ],
        out_specs=pl.BlockSpec((2, _RB, DH), lambda i: (0, i, 0)),
        out_shape=jax.ShapeDtypeStruct((2, NP, DH), jnp.float32),
    )(s, y, p0, p1, b, w)


def _final_body(s_ref, y_ref, p0_ref, p1_ref, b_ref, o_ref):
    dinv = lax.rsqrt(p0_ref[...] + p1_ref[...] + 1.0)
    b = b_ref[...]
    o_ref[:, :DH] = (s_ref[0] + y_ref[0]) * dinv + b[:, :DH]
    o_ref[:, DH:] = (s_ref[1] + y_ref[1]) * dinv + b[:, DH:]


def _final(s, y, p0, p1, b):
    return pl.pallas_call(
        _final_body,
        grid=(_NG,),
        in_specs=[
            pl.BlockSpec((2, _RB, DH), lambda i: (0, i, 0)),
            pl.BlockSpec((2, _RB, DH), lambda i: (0, i, 0)),
            pl.BlockSpec((_RB, 1), lambda i: (i, 0)),
            pl.BlockSpec((_RB, 1), lambda i: (i, 0)),
            pl.BlockSpec((1, D), lambda i: (0, 0)),
        ],
        out_specs=pl.BlockSpec((_RB, D), lambda i: (i, 0)),
        out_shape=jax.ShapeDtypeStruct((NN, D), jnp.float32),
    )(s, y, p0, p1, b)


def kernel(X, edges, W1, b1, W2, b2):
    src = edges[0].astype(jnp.int32)
    dst = edges[1].astype(jnp.int32)
    npad = EP - NE
    pad_idx = NN + (jnp.arange(npad, dtype=jnp.int32) % (NP - NN))
    src_p = jnp.concatenate([src, pad_idx])
    dst_p = jnp.concatenate([dst, pad_idx])
    srcs2 = jnp.concatenate([src_p, src_p + NP])
    dst3 = dst_p.reshape(NSUB, NCH, K)
    dst_deg = dst_p.reshape(2 * NSUB, NCHD, K)

    zeros_np = jnp.zeros((NP,), jnp.float32)
    b1r = b1.reshape(1, D)
    b2r = b2.reshape(1, D)

    degf = _degree_kernel(dst_deg, zeros_np)
    p0 = degf[:NP].reshape(NP, 1)
    p1 = degf[NP:].reshape(NP, 1)

    y1 = _mm_scale(X, W1, p0, p1)
    s1 = _segsum_kernel(y1.reshape(2 * NP, DH), srcs2, dst3)
    y2 = _combine_mm(s1.reshape(2, NP, DH), y1, p0, p1, b1r, W2)
    s2 = _segsum_kernel(y2.reshape(2 * NP, DH), srcs2, dst3)
    return _final(s2.reshape(2, NP, DH), y2, p0, p1, b2r)


# speedup vs baseline: 1.1176x; 1.0028x over previous
import functools

import jax
import jax.numpy as jnp
from jax import lax
from jax.experimental import pallas as pl
from jax.experimental.pallas import tpu as pltpu
from jax.experimental.pallas import tpu_sc as plsc

NN = 10000
NP = 10240
NE = 160000
EP = 163840
D = 256
DH = 128
NSUB = 16
NCORE = 2
K = 128
EPS = EP // NSUB
EPW = EP // (NSUB * NCORE)
STRIPE = NP // NSUB

_mesh = plsc.VectorSubcoreMesh(core_axis_name="c", subcore_axis_name="s")


NCHD = EPW // K
NCH = EPS // K


@functools.partial(
    pl.kernel,
    mesh=_mesh,
    out_type=jax.ShapeDtypeStruct((NCORE * NP,), jnp.float32),
    scratch_types=[
        pltpu.VMEM((NCHD, K), jnp.int32),
        pltpu.VMEM((K,), jnp.float32),
        pltpu.VMEM_SHARED((NP,), jnp.float32),
        pltpu.SemaphoreType.DMA,
    ],
)
def _degree_kernel(dst_hbm, zeros_hbm, out_hbm, idx_v, ones_v, acc_sh, ssem):
    cid = lax.axis_index("c")
    sid = lax.axis_index("s")
    wid = cid * NSUB + sid
    pltpu.sync_copy(dst_hbm.at[wid], idx_v)
    for j in range(K // 16):
        ones_v[pl.ds(16 * j, 16)] = jnp.ones((16,), jnp.float32)

    @pl.when(sid == 0)
    def _zero():
        pltpu.sync_copy(zeros_hbm, acc_sh)

    plsc.subcore_barrier()

    def fire(i, carry):
        pltpu.async_copy(ones_v, acc_sh.at[idx_v.at[i]], ssem, add=True)
        return carry

    def drain(i, carry):
        pltpu.make_async_copy(ones_v, acc_sh.at[idx_v.at[0]], ssem).wait()
        return carry

    lax.fori_loop(0, NCHD, fire, 0)
    lax.fori_loop(0, NCHD, drain, 0)
    plsc.subcore_barrier()
    pltpu.sync_copy(
        acc_sh.at[pl.ds(sid * STRIPE, STRIPE)],
        out_hbm.at[pl.ds(cid * NP + sid * STRIPE, STRIPE)],
    )


@functools.partial(
    pl.kernel,
    mesh=_mesh,
    out_type=jax.ShapeDtypeStruct((NCORE * NP, DH), jnp.float32),
    scratch_types=[
        pltpu.VMEM((K,), jnp.int32),
        pltpu.VMEM((K,), jnp.int32),
        pltpu.VMEM((NCH, K), jnp.int32),
        pltpu.VMEM((K, DH), jnp.float32),
        pltpu.VMEM((K, DH), jnp.float32),
        pltpu.VMEM_SHARED((NP, DH), jnp.float32),
        pltpu.SemaphoreType.DMA,
        pltpu.SemaphoreType.DMA,
        pltpu.SemaphoreType.DMA,
        pltpu.SemaphoreType.DMA,
        pltpu.SemaphoreType.DMA,
        pltpu.SemaphoreType.DMA,
    ],
)
def _segsum_kernel(y_hbm, srcs_hbm, dst_hbm, out_hbm,
                   src0, src1, dst_v, rows0, rows1,
                   acc_sh, isem0, isem1, gsem0, gsem1, ssem0, ssem1):
    cid = lax.axis_index("c")
    sid = lax.axis_index("s")
    wid = cid * NSUB + sid
    sbase = wid * EPS
    pltpu.async_copy(dst_hbm.at[sid], dst_v, isem0)
    zv = jnp.zeros((16,), jnp.float32)

    def _zrow(r, carry):
        for c in range(DH // 16):
            rows0[r, pl.ds(c * 16, 16)] = zv
        return carry

    lax.fori_loop(0, K, _zrow, 0)
    for t in range(STRIPE // K):
        pltpu.async_copy(rows0, acc_sh.at[pl.ds(sid * STRIPE + t * K, K), :], ssem0)
    pltpu.make_async_copy(dst_hbm.at[sid], dst_v, isem0).wait()
    for t in range(STRIPE // K):
        pltpu.make_async_copy(rows0, acc_sh.at[pl.ds(sid * STRIPE, K), :], ssem0).wait()
    plsc.subcore_barrier()
    bufs = ((src0, isem0, rows0, gsem0, ssem0),
            (src1, isem1, rows1, gsem1, ssem1))

    def start_idx(j, b):
        sv, isem = bufs[b][0], bufs[b][1]
        j = jnp.minimum(j, NCH - 1)
        pltpu.async_copy(srcs_hbm.at[pl.ds(sbase + j * K, K)], sv, isem)

    def wait_idx(b):
        sv, isem = bufs[b][0], bufs[b][1]
        pltpu.make_async_copy(srcs_hbm.at[pl.ds(sbase, K)], sv, isem).wait()

    def start_gather(b):
        sv, rv, gsem = bufs[b][0], bufs[b][2], bufs[b][3]
        pltpu.async_copy(y_hbm.at[sv], rv, gsem)

    def wait_gather(b):
        sv, rv, gsem = bufs[b][0], bufs[b][2], bufs[b][3]
        pltpu.make_async_copy(y_hbm.at[sv], rv, gsem).wait()

    def start_scatter(i, b):
        rv, ssem = bufs[b][2], bufs[b][4]
        pltpu.async_copy(rv, acc_sh.at[dst_v.at[i]], ssem, add=True)

    def wait_scatter(b):
        rv, ssem = bufs[b][2], bufs[b][4]
        pltpu.make_async_copy(rv, acc_sh.at[dst_v.at[0]], ssem).wait()

    start_idx(0, 0)
    start_idx(1, 1)
    wait_idx(0)
    start_gather(0)
    wait_gather(0)
    start_scatter(0, 0)
    start_idx(2, 0)
    wait_idx(1)
    start_gather(1)

    def step(i, b):
        nb = 1 - b
        wait_gather(b)
        start_scatter(i, b)
        start_idx(i + 2, b)
        wait_scatter(nb)
        wait_idx(nb)
        start_gather(nb)

    def pair(g, carry):
        step(2 * g + 1, 1)
        step(2 * g + 2, 0)
        return carry

    lax.fori_loop(0, (NCH - 2) // 2, pair, 0)
    wait_gather(1)
    start_scatter(NCH - 1, 1)
    wait_scatter(0)
    wait_scatter(1)
    wait_idx(0)
    plsc.subcore_barrier()
    pltpu.sync_copy(
        acc_sh.at[pl.ds(sid * STRIPE, STRIPE), :],
        out_hbm.at[pl.ds(cid * NP + sid * STRIPE, STRIPE), :],
    )


_RB = 1000
_NG = NN // _RB


def _mm_scale_body(x_ref, w_ref, p0_ref, p1_ref, y_ref):
    dinv = lax.rsqrt(p0_ref[...] + p1_ref[...] + 1.0)
    y = jnp.dot(x_ref[...], w_ref[...], preferred_element_type=jnp.float32)
    y = y * dinv
    y_ref[0] = y[:, :DH]
    y_ref[1] = y[:, DH:]


def _mm_scale(xp, w, p0, p1):
    return pl.pallas_call(
        _mm_scale_body,
        grid=(_NG,),
        in_specs=[
            pl.BlockSpec((_RB, D), lambda i: (i, 0)),
            pl.BlockSpec((D, D), lambda i: (0, 0)),
            pl.BlockSpec((_RB, 1), lambda i: (i, 0)),
            pl.BlockSpec((_RB, 1), lambda i: (i, 0)),
        ],
        out_specs=pl.BlockSpec((2, _RB, DH), lambda i: (0, i, 0)),
        out_shape=jax.ShapeDtypeStruct((2, NP, DH), jnp.float32),
    )(xp, w, p0, p1)


def _combine_mm_body(s_ref, y_ref, p0_ref, p1_ref, b_ref, w_ref, o_ref):
    dinv = lax.rsqrt(p0_ref[...] + p1_ref[...] + 1.0)
    b = b_ref[...]
    h0 = jnp.maximum((s_ref[0] + y_ref[0]) * dinv + b[:, :DH], 0.0)
    h1 = jnp.maximum((s_ref[1] + y_ref[1]) * dinv + b[:, DH:], 0.0)
    y2 = jnp.dot(h0, w_ref[:DH, :], preferred_element_type=jnp.float32)
    y2 = y2 + jnp.dot(h1, w_ref[DH:, :], preferred_element_type=jnp.float32)
    y2 = y2 * dinv
    o_ref[0] = y2[:, :DH]
    o_ref[1] = y2[:, DH:]


def _combine_mm(s, y, p0, p1, b, w):
    return pl.pallas_call(
        _combine_mm_body,
        grid=(_NG,),
        in_specs=[
            pl.BlockSpec((2, _RB, DH), lambda i: (0, i, 0)),
            pl.BlockSpec((2, _RB, DH), lambda i: (0, i, 0)),
            pl.BlockSpec((_RB, 1), lambda i: (i, 0)),
            pl.BlockSpec((_RB, 1), lambda i: (i, 0)),
            pl.BlockSpec((1, D), lambda i: (0, 0)),
            pl.BlockSpec((D, D), lambda i: (0, 0)),
        ],
        out_specs=pl.BlockSpec((2, _RB, DH), lambda i: (0, i, 0)),
        out_shape=jax.ShapeDtypeStruct((2, NP, DH), jnp.float32),
    )(s, y, p0, p1, b, w)


def _final_body(s_ref, y_ref, p0_ref, p1_ref, b_ref, o_ref):
    dinv = lax.rsqrt(p0_ref[...] + p1_ref[...] + 1.0)
    b = b_ref[...]
    o_ref[:, :DH] = (s_ref[0] + y_ref[0]) * dinv + b[:, :DH]
    o_ref[:, DH:] = (s_ref[1] + y_ref[1]) * dinv + b[:, DH:]


def _final(s, y, p0, p1, b):
    return pl.pallas_call(
        _final_body,
        grid=(_NG,),
        in_specs=[
            pl.BlockSpec((2, _RB, DH), lambda i: (0, i, 0)),
            pl.BlockSpec((2, _RB, DH), lambda i: (0, i, 0)),
            pl.BlockSpec((_RB, 1), lambda i: (i, 0)),
            pl.BlockSpec((_RB, 1), lambda i: (i, 0)),
            pl.BlockSpec((1, D), lambda i: (0, 0)),
        ],
        out_specs=pl.BlockSpec((_RB, D), lambda i: (i, 0)),
        out_shape=jax.ShapeDtypeStruct((NN, D), jnp.float32),
    )(s, y, p0, p1, b)


def kernel(X, edges, W1, b1, W2, b2):
    src = edges[0].astype(jnp.int32)
    dst = edges[1].astype(jnp.int32)
    npad = EP - NE
    pad_idx = NN + (jnp.arange(npad, dtype=jnp.int32) % (NP - NN))
    src_p = jnp.concatenate([src, pad_idx])
    dst_p = jnp.concatenate([dst, pad_idx])
    srcs2 = jnp.concatenate([src_p, src_p + NP])
    dst3 = dst_p.reshape(NSUB, NCH, K)
    dst_deg = dst_p.reshape(2 * NSUB, NCHD, K)

    zeros_np = jnp.zeros((NP,), jnp.float32)
    b1r = b1.reshape(1, D)
    b2r = b2.reshape(1, D)

    degf = _degree_kernel(dst_deg, zeros_np)
    p0 = degf[:NP].reshape(NP, 1)
    p1 = degf[NP:].reshape(NP, 1)

    y1 = _mm_scale(X, W1, p0, p1)
    s1 = _segsum_kernel(y1.reshape(2 * NP, DH), srcs2, dst3)
    y2 = _combine_mm(s1.reshape(2, NP, DH), y1, p0, p1, b1r, W2)
    s2 = _segsum_kernel(y2.reshape(2 * NP, DH), srcs2, dst3)
    return _final(s2.reshape(2, NP, DH), y2, p0, p1, b2r)


# speedup vs baseline: 1.1177x; 1.0001x over previous
import functools

import jax
import jax.numpy as jnp
from jax import lax
from jax.experimental import pallas as pl
from jax.experimental.pallas import tpu as pltpu
from jax.experimental.pallas import tpu_sc as plsc

NN = 10000
NP = 10240
NE = 160000
EP = 163840
D = 256
DH = 128
NSUB = 16
NCORE = 2
K = 128
EPS = EP // NSUB
EPW = EP // (NSUB * NCORE)
STRIPE = NP // NSUB

_mesh = plsc.VectorSubcoreMesh(core_axis_name="c", subcore_axis_name="s")


NCHD = EPW // K
NCH = EPS // K


@functools.partial(
    pl.kernel,
    mesh=_mesh,
    out_type=jax.ShapeDtypeStruct((NCORE * NP,), jnp.float32),
    scratch_types=[
        pltpu.VMEM((NCHD, K), jnp.int32),
        pltpu.VMEM((K,), jnp.float32),
        pltpu.VMEM_SHARED((NP,), jnp.float32),
        pltpu.SemaphoreType.DMA,
    ],
)
def _degree_kernel(dst_hbm, zeros_hbm, out_hbm, idx_v, ones_v, acc_sh, ssem):
    cid = lax.axis_index("c")
    sid = lax.axis_index("s")
    wid = cid * NSUB + sid
    pltpu.sync_copy(dst_hbm.at[wid], idx_v)
    for j in range(K // 16):
        ones_v[pl.ds(16 * j, 16)] = jnp.ones((16,), jnp.float32)

    @pl.when(sid == 0)
    def _zero():
        pltpu.sync_copy(zeros_hbm, acc_sh)

    plsc.subcore_barrier()

    def fire(i, carry):
        pltpu.async_copy(ones_v, acc_sh.at[idx_v.at[i]], ssem, add=True)
        return carry

    def drain(i, carry):
        pltpu.make_async_copy(ones_v, acc_sh.at[idx_v.at[0]], ssem).wait()
        return carry

    lax.fori_loop(0, NCHD, fire, 0)
    lax.fori_loop(0, NCHD, drain, 0)
    plsc.subcore_barrier()
    pltpu.sync_copy(
        acc_sh.at[pl.ds(sid * STRIPE, STRIPE)],
        out_hbm.at[pl.ds(cid * NP + sid * STRIPE, STRIPE)],
    )


@functools.partial(
    pl.kernel,
    mesh=_mesh,
    out_type=jax.ShapeDtypeStruct((NCORE * NP, DH), jnp.float32),
    scratch_types=[
        pltpu.VMEM((K,), jnp.int32),
        pltpu.VMEM((K,), jnp.int32),
        pltpu.VMEM((NCH, K), jnp.int32),
        pltpu.VMEM((K, DH), jnp.float32),
        pltpu.VMEM((K, DH), jnp.float32),
        pltpu.VMEM_SHARED((NP, DH), jnp.float32),
        pltpu.SemaphoreType.DMA,
        pltpu.SemaphoreType.DMA,
        pltpu.SemaphoreType.DMA,
        pltpu.SemaphoreType.DMA,
        pltpu.SemaphoreType.DMA,
        pltpu.SemaphoreType.DMA,
    ],
)
def _segsum_kernel(y_hbm, srcs_hbm, dst_hbm, out_hbm,
                   src0, src1, dst_v, rows0, rows1,
                   acc_sh, isem0, isem1, gsem0, gsem1, ssem0, ssem1):
    cid = lax.axis_index("c")
    sid = lax.axis_index("s")
    wid = cid * NSUB + sid
    sbase = wid * EPS
    pltpu.async_copy(dst_hbm.at[sid], dst_v, isem0)
    zv = jnp.zeros((16,), jnp.float32)

    def _zrow(r, carry):
        for c in range(DH // 16):
            rows0[r, pl.ds(c * 16, 16)] = zv
        return carry

    lax.fori_loop(0, K, _zrow, 0)
    for t in range(STRIPE // K):
        pltpu.async_copy(rows0, acc_sh.at[pl.ds(sid * STRIPE + t * K, K), :], ssem0)
    pltpu.make_async_copy(dst_hbm.at[sid], dst_v, isem0).wait()
    for t in range(STRIPE // K):
        pltpu.make_async_copy(rows0, acc_sh.at[pl.ds(sid * STRIPE, K), :], ssem0).wait()
    plsc.subcore_barrier()
    bufs = ((src0, isem0, rows0, gsem0, ssem0),
            (src1, isem1, rows1, gsem1, ssem1))

    def start_idx(j, b):
        sv, isem = bufs[b][0], bufs[b][1]
        j = jnp.minimum(j, NCH - 1)
        pltpu.async_copy(srcs_hbm.at[pl.ds(sbase + j * K, K)], sv, isem)

    def wait_idx(b):
        sv, isem = bufs[b][0], bufs[b][1]
        pltpu.make_async_copy(srcs_hbm.at[pl.ds(sbase, K)], sv, isem).wait()

    def start_gather(b):
        sv, rv, gsem = bufs[b][0], bufs[b][2], bufs[b][3]
        pltpu.async_copy(y_hbm.at[sv.at[pl.ds(0, K // 2)]],
                         rv.at[pl.ds(0, K // 2), :], gsem)
        pltpu.async_copy(y_hbm.at[sv.at[pl.ds(K // 2, K // 2)]],
                         rv.at[pl.ds(K // 2, K // 2), :], gsem)

    def wait_gather(b):
        sv, rv, gsem = bufs[b][0], bufs[b][2], bufs[b][3]
        pltpu.make_async_copy(y_hbm.at[sv.at[pl.ds(0, K // 2)]],
                              rv.at[pl.ds(0, K // 2), :], gsem).wait()
        pltpu.make_async_copy(y_hbm.at[sv.at[pl.ds(K // 2, K // 2)]],
                              rv.at[pl.ds(K // 2, K // 2), :], gsem).wait()

    def start_scatter(i, b):
        rv, ssem = bufs[b][2], bufs[b][4]
        pltpu.async_copy(rv, acc_sh.at[dst_v.at[i]], ssem, add=True)

    def wait_scatter(b):
        rv, ssem = bufs[b][2], bufs[b][4]
        pltpu.make_async_copy(rv, acc_sh.at[dst_v.at[0]], ssem).wait()

    start_idx(0, 0)
    start_idx(1, 1)
    wait_idx(0)
    start_gather(0)
    wait_gather(0)
    start_scatter(0, 0)
    start_idx(2, 0)
    wait_idx(1)
    start_gather(1)

    def step(i, b):
        nb = 1 - b
        wait_gather(b)
        start_scatter(i, b)
        start_idx(i + 2, b)
        wait_scatter(nb)
        wait_idx(nb)
        start_gather(nb)

    def pair(g, carry):
        step(2 * g + 1, 1)
        step(2 * g + 2, 0)
        return carry

    lax.fori_loop(0, (NCH - 2) // 2, pair, 0)
    wait_gather(1)
    start_scatter(NCH - 1, 1)
    wait_scatter(0)
    wait_scatter(1)
    wait_idx(0)
    plsc.subcore_barrier()
    pltpu.sync_copy(
        acc_sh.at[pl.ds(sid * STRIPE, STRIPE), :],
        out_hbm.at[pl.ds(cid * NP + sid * STRIPE, STRIPE), :],
    )


_RB = 1000
_NG = NN // _RB


def _mm_scale_body(x_ref, w_ref, p0_ref, p1_ref, y_ref):
    dinv = lax.rsqrt(p0_ref[...] + p1_ref[...] + 1.0)
    y = jnp.dot(x_ref[...], w_ref[...], preferred_element_type=jnp.float32)
    y = y * dinv
    y_ref[0] = y[:, :DH]
    y_ref[1] = y[:, DH:]


def _mm_scale(xp, w, p0, p1):
    return pl.pallas_call(
        _mm_scale_body,
        grid=(_NG,),
        in_specs=[
            pl.BlockSpec((_RB, D), lambda i: (i, 0)),
            pl.BlockSpec((D, D), lambda i: (0, 0)),
            pl.BlockSpec((_RB, 1), lambda i: (i, 0)),
            pl.BlockSpec((_RB, 1), lambda i: (i, 0)),
        ],
        out_specs=pl.BlockSpec((2, _RB, DH), lambda i: (0, i, 0)),
        out_shape=jax.ShapeDtypeStruct((2, NP, DH), jnp.float32),
    )(xp, w, p0, p1)


def _combine_mm_body(s_ref, y_ref, p0_ref, p1_ref, b_ref, w_ref, o_ref):
    dinv = lax.rsqrt(p0_ref[...] + p1_ref[...] + 1.0)
    b = b_ref[...]
    h0 = jnp.maximum((s_ref[0] + y_ref[0]) * dinv + b[:, :DH], 0.0)
    h1 = jnp.maximum((s_ref[1] + y_ref[1]) * dinv + b[:, DH:], 0.0)
    y2 = jnp.dot(h0, w_ref[:DH, :], preferred_element_type=jnp.float32)
    y2 = y2 + jnp.dot(h1, w_ref[DH:, :], preferred_element_type=jnp.float32)
    y2 = y2 * dinv
    o_ref[0] = y2[:, :DH]
    o_ref[1] = y2[:, DH:]


def _combine_mm(s, y, p0, p1, b, w):
    return pl.pallas_call(
        _combine_mm_body,
        grid=(_NG,),
        in_specs=[
            pl.BlockSpec((2, _RB, DH), lambda i: (0, i, 0)),
            pl.BlockSpec((2, _RB, DH), lambda i: (0, i, 0)),
            pl.BlockSpec((_RB, 1), lambda i: (i, 0)),
            pl.BlockSpec((_RB, 1), lambda i: (i, 0)),
            pl.BlockSpec((1, D), lambda i: (0, 0)),
            pl.BlockSpec((D, D), lambda i: (0, 0)),
        ],
        out_specs=pl.BlockSpec((2, _RB, DH), lambda i: (0, i, 0)),
        out_shape=jax.ShapeDtypeStruct((2, NP, DH), jnp.float32),
    )(s, y, p0, p1, b, w)


def _final_body(s_ref, y_ref, p0_ref, p1_ref, b_ref, o_ref):
    dinv = lax.rsqrt(p0_ref[...] + p1_ref[...] + 1.0)
    b = b_ref[...]
    o_ref[:, :DH] = (s_ref[0] + y_ref[0]) * dinv + b[:, :DH]
    o_ref[:, DH:] = (s_ref[1] + y_ref[1]) * dinv + b[:, DH:]


def _final(s, y, p0, p1, b):
    return pl.pallas_call(
        _final_body,
        grid=(_NG,),
        in_specs=[
            pl.BlockSpec((2, _RB, DH), lambda i: (0, i, 0)),
            pl.BlockSpec((2, _RB, DH), lambda i: (0, i, 0)),
            pl.BlockSpec((_RB, 1), lambda i: (i, 0)),
            pl.BlockSpec((_RB, 1), lambda i: (i, 0)),
            pl.BlockSpec((1, D), lambda i: (0, 0)),
        ],
        out_specs=pl.BlockSpec((_RB, D), lambda i: (i, 0)),
        out_shape=jax.ShapeDtypeStruct((NN, D), jnp.float32),
    )(s, y, p0, p1, b)


def kernel(X, edges, W1, b1, W2, b2):
    src = edges[0].astype(jnp.int32)
    dst = edges[1].astype(jnp.int32)
    npad = EP - NE
    pad_idx = NN + (jnp.arange(npad, dtype=jnp.int32) % (NP - NN))
    src_p = jnp.concatenate([src, pad_idx])
    dst_p = jnp.concatenate([dst, pad_idx])
    srcs2 = jnp.concatenate([src_p, src_p + NP])
    dst3 = dst_p.reshape(NSUB, NCH, K)
    dst_deg = dst_p.reshape(2 * NSUB, NCHD, K)

    zeros_np = jnp.zeros((NP,), jnp.float32)
    b1r = b1.reshape(1, D)
    b2r = b2.reshape(1, D)

    degf = _degree_kernel(dst_deg, zeros_np)
    p0 = degf[:NP].reshape(NP, 1)
    p1 = degf[NP:].reshape(NP, 1)

    y1 = _mm_scale(X, W1, p0, p1)
    s1 = _segsum_kernel(y1.reshape(2 * NP, DH), srcs2, dst3)
    y2 = _combine_mm(s1.reshape(2, NP, DH), y1, p0, p1, b1r, W2)
    s2 = _segsum_kernel(y2.reshape(2 * NP, DH), srcs2, dst3)
    return _final(s2.reshape(2, NP, DH), y2, p0, p1, b2r)


# speedup vs baseline: 1.1182x; 1.0004x over previous
import functools

import jax
import jax.numpy as jnp
from jax import lax
from jax.experimental import pallas as pl
from jax.experimental.pallas import tpu as pltpu
from jax.experimental.pallas import tpu_sc as plsc

NN = 10000
NP = 10240
NE = 160000
EP = 163840
D = 256
DH = 128
NSUB = 16
NCORE = 2
K = 128
EPS = EP // NSUB
EPW = EP // (NSUB * NCORE)
STRIPE = NP // NSUB

_mesh = plsc.VectorSubcoreMesh(core_axis_name="c", subcore_axis_name="s")


NCHD = EPW // K
NCH = EPS // K


@functools.partial(
    pl.kernel,
    mesh=_mesh,
    out_type=jax.ShapeDtypeStruct((NCORE * NP,), jnp.float32),
    scratch_types=[
        pltpu.VMEM((NCHD, K), jnp.int32),
        pltpu.VMEM((K,), jnp.float32),
        pltpu.VMEM_SHARED((NP,), jnp.float32),
        pltpu.SemaphoreType.DMA,
    ],
)
def _degree_kernel(dst_hbm, zeros_hbm, out_hbm, idx_v, ones_v, acc_sh, ssem):
    cid = lax.axis_index("c")
    sid = lax.axis_index("s")
    wid = cid * NSUB + sid
    pltpu.sync_copy(dst_hbm.at[wid], idx_v)
    for j in range(K // 16):
        ones_v[pl.ds(16 * j, 16)] = jnp.ones((16,), jnp.float32)

    @pl.when(sid == 0)
    def _zero():
        pltpu.sync_copy(zeros_hbm, acc_sh)

    plsc.subcore_barrier()

    def fire(i, carry):
        pltpu.async_copy(ones_v, acc_sh.at[idx_v.at[i]], ssem, add=True)
        return carry

    def drain(i, carry):
        pltpu.make_async_copy(ones_v, acc_sh.at[idx_v.at[0]], ssem).wait()
        return carry

    lax.fori_loop(0, NCHD, fire, 0)
    lax.fori_loop(0, NCHD, drain, 0)
    plsc.subcore_barrier()
    pltpu.sync_copy(
        acc_sh.at[pl.ds(sid * STRIPE, STRIPE)],
        out_hbm.at[pl.ds(cid * NP + sid * STRIPE, STRIPE)],
    )


@functools.partial(
    pl.kernel,
    mesh=_mesh,
    out_type=jax.ShapeDtypeStruct((NCORE * NP, DH), jnp.float32),
    scratch_types=[
        pltpu.VMEM((K,), jnp.int32),
        pltpu.VMEM((K,), jnp.int32),
        pltpu.VMEM((NCH, K), jnp.int32),
        pltpu.VMEM((K, DH), jnp.float32),
        pltpu.VMEM((K, DH), jnp.float32),
        pltpu.VMEM_SHARED((NP, DH), jnp.float32),
        pltpu.SemaphoreType.DMA,
        pltpu.SemaphoreType.DMA,
        pltpu.SemaphoreType.DMA,
        pltpu.SemaphoreType.DMA,
        pltpu.SemaphoreType.DMA,
        pltpu.SemaphoreType.DMA,
    ],
)
def _segsum_kernel(y_hbm, srcs_hbm, dst_hbm, out_hbm,
                   src0, src1, dst_v, rows0, rows1,
                   acc_sh, isem0, isem1, gsem0, gsem1, ssem0, ssem1):
    cid = lax.axis_index("c")
    sid = lax.axis_index("s")
    wid = cid * NSUB + sid
    sbase = wid * EPS
    pltpu.async_copy(dst_hbm.at[sid], dst_v, isem0)
    zv = jnp.zeros((16,), jnp.float32)

    def _zrow(r, carry):
        for c in range(DH // 16):
            rows0[r, pl.ds(c * 16, 16)] = zv
        return carry

    lax.fori_loop(0, K, _zrow, 0)
    for t in range(STRIPE // K):
        pltpu.async_copy(rows0, acc_sh.at[pl.ds(sid * STRIPE + t * K, K), :], ssem0)
    pltpu.make_async_copy(dst_hbm.at[sid], dst_v, isem0).wait()
    for t in range(STRIPE // K):
        pltpu.make_async_copy(rows0, acc_sh.at[pl.ds(sid * STRIPE, K), :], ssem0).wait()
    plsc.subcore_barrier()
    bufs = ((src0, isem0, rows0, gsem0, ssem0),
            (src1, isem1, rows1, gsem1, ssem1))

    def start_idx(j, b):
        sv, isem = bufs[b][0], bufs[b][1]
        j = jnp.minimum(j, NCH - 1)
        pltpu.async_copy(srcs_hbm.at[pl.ds(sbase + j * K, K)], sv, isem)

    def wait_idx(b):
        sv, isem = bufs[b][0], bufs[b][1]
        pltpu.make_async_copy(srcs_hbm.at[pl.ds(sbase, K)], sv, isem).wait()

    def start_gather(b):
        sv, rv, gsem = bufs[b][0], bufs[b][2], bufs[b][3]
        pltpu.async_copy(y_hbm.at[sv], rv, gsem)

    def wait_gather(b):
        sv, rv, gsem = bufs[b][0], bufs[b][2], bufs[b][3]
        pltpu.make_async_copy(y_hbm.at[sv], rv, gsem).wait()

    def start_scatter(i, b):
        rv, ssem = bufs[b][2], bufs[b][4]
        pltpu.async_copy(rv, acc_sh.at[dst_v.at[i]], ssem, add=True)

    def wait_scatter(b):
        rv, ssem = bufs[b][2], bufs[b][4]
        pltpu.make_async_copy(rv, acc_sh.at[dst_v.at[0]], ssem).wait()

    start_idx(0, 0)
    start_idx(1, 1)
    wait_idx(0)
    start_gather(0)
    wait_gather(0)
    start_scatter(0, 0)
    start_idx(2, 0)
    wait_idx(1)
    start_gather(1)

    def step(i, b):
        nb = 1 - b
        wait_gather(b)
        start_scatter(i, b)
        start_idx(i + 2, b)
        wait_scatter(nb)
        wait_idx(nb)
        start_gather(nb)

    def pair(g, carry):
        step(2 * g + 1, 1)
        step(2 * g + 2, 0)
        return carry

    lax.fori_loop(0, (NCH - 2) // 2, pair, 0)
    wait_gather(1)
    start_scatter(NCH - 1, 1)
    wait_scatter(0)
    wait_scatter(1)
    wait_idx(0)
    plsc.subcore_barrier()
    pltpu.sync_copy(
        acc_sh.at[pl.ds(sid * STRIPE, STRIPE), :],
        out_hbm.at[pl.ds(cid * NP + sid * STRIPE, STRIPE), :],
    )


_RB = 1000
_NG = NN // _RB


def _mm_scale_body(x_ref, w_ref, p0_ref, p1_ref, y_ref):
    dinv = lax.rsqrt(p0_ref[...] + p1_ref[...] + 1.0)
    y = jnp.dot(x_ref[...], w_ref[...], preferred_element_type=jnp.float32)
    y = y * dinv
    y_ref[0] = y[:, :DH]
    y_ref[1] = y[:, DH:]


def _mm_scale(xp, w, p0, p1):
    return pl.pallas_call(
        _mm_scale_body,
        grid=(_NG,),
        in_specs=[
            pl.BlockSpec((_RB, D), lambda i: (i, 0)),
            pl.BlockSpec((D, D), lambda i: (0, 0)),
            pl.BlockSpec((_RB, 1), lambda i: (i, 0)),
            pl.BlockSpec((_RB, 1), lambda i: (i, 0)),
        ],
        out_specs=pl.BlockSpec((2, _RB, DH), lambda i: (0, i, 0)),
        out_shape=jax.ShapeDtypeStruct((2, NP, DH), jnp.float32),
    )(xp, w, p0, p1)


def _combine_mm_body(s_ref, y_ref, p0_ref, p1_ref, b_ref, w_ref, o_ref):
    dinv = lax.rsqrt(p0_ref[...] + p1_ref[...] + 1.0)
    b = b_ref[...]
    h0 = jnp.maximum((s_ref[0] + y_ref[0]) * dinv + b[:, :DH], 0.0)
    h1 = jnp.maximum((s_ref[1] + y_ref[1]) * dinv + b[:, DH:], 0.0)
    y2 = jnp.dot(h0, w_ref[:DH, :], preferred_element_type=jnp.float32)
    y2 = y2 + jnp.dot(h1, w_ref[DH:, :], preferred_element_type=jnp.float32)
    y2 = y2 * dinv
    o_ref[0] = y2[:, :DH]
    o_ref[1] = y2[:, DH:]


def _combine_mm(s, y, p0, p1, b, w):
    return pl.pallas_call(
        _combine_mm_body,
        grid=(_NG,),
        in_specs=[
            pl.BlockSpec((2, _RB, DH), lambda i: (0, i, 0)),
            pl.BlockSpec((2, _RB, DH), lambda i: (0, i, 0)),
            pl.BlockSpec((_RB, 1), lambda i: (i, 0)),
            pl.BlockSpec((_RB, 1), lambda i: (i, 0)),
            pl.BlockSpec((1, D), lambda i: (0, 0)),
            pl.BlockSpec((D, D), lambda i: (0, 0)),
        ],
        out_specs=pl.BlockSpec((2, _RB, DH), lambda i: (0, i, 0)),
        out_shape=jax.ShapeDtypeStruct((2, NP, DH), jnp.float32),
    )(s, y, p0, p1, b, w)


def _final_body(s_ref, y_ref, p0_ref, p1_ref, b_ref, o_ref):
    dinv = lax.rsqrt(p0_ref[...] + p1_ref[...] + 1.0)
    b = b_ref[...]
    o_ref[:, :DH] = (s_ref[0] + y_ref[0]) * dinv + b[:, :DH]
    o_ref[:, DH:] = (s_ref[1] + y_ref[1]) * dinv + b[:, DH:]


def _final(s, y, p0, p1, b):
    return pl.pallas_call(
        _final_body,
        grid=(_NG,),
        in_specs=[
            pl.BlockSpec((2, _RB, DH), lambda i: (0, i, 0)),
            pl.BlockSpec((2, _RB, DH), lambda i: (0, i, 0)),
            pl.BlockSpec((_RB, 1), lambda i: (i, 0)),
            pl.BlockSpec((_RB, 1), lambda i: (i, 0)),
            pl.BlockSpec((1, D), lambda i: (0, 0)),
        ],
        out_specs=pl.BlockSpec((_RB, D), lambda i: (i, 0)),
        out_shape=jax.ShapeDtypeStruct((NN, D), jnp.float32),
    )(s, y, p0, p1, b)


def kernel(X, edges, W1, b1, W2, b2):
    src = edges[0].astype(jnp.int32)
    dst = edges[1].astype(jnp.int32)
    npad = EP - NE
    pad_idx = NN + (jnp.arange(npad, dtype=jnp.int32) % (NP - NN))
    src_p = jnp.concatenate([src, pad_idx])
    dst_p = jnp.concatenate([dst, pad_idx])
    srcs2 = jnp.concatenate([src_p, src_p + NP])
    dst3 = dst_p.reshape(NSUB, NCH, K)
    dst_deg = dst_p.reshape(2 * NSUB, NCHD, K)

    zeros_np = jnp.zeros((NP,), jnp.float32)
    b1r = b1.reshape(1, D)
    b2r = b2.reshape(1, D)

    degf = _degree_kernel(dst_deg, zeros_np)
    p0 = degf[:NP].reshape(NP, 1)
    p1 = degf[NP:].reshape(NP, 1)

    y1 = _mm_scale(X, W1, p0, p1)
    s1 = _segsum_kernel(y1.reshape(2 * NP, DH), srcs2, dst3)
    y2 = _combine_mm(s1.reshape(2, NP, DH), y1, p0, p1, b1r, W2)
    s2 = _segsum_kernel(y2.reshape(2 * NP, DH), srcs2, dst3)
    return _final(s2.reshape(2, NP, DH), y2, p0, p1, b2r)
